# baseline (speedup 1.0000x reference)
import functools

import jax
import jax.numpy as jnp
from jax import lax
from jax.experimental import pallas as pl
from jax.experimental.pallas import tpu as pltpu
from jax.experimental.pallas import tpu_sc as plsc

_N = 262144
_C = 64
_NUM_WRITE_OPS = 8
_K = 4
_NS_ROWS = _N // _K

_NC = 2
_NS = 16
_NW = _NC * _NS
_L = 16

_ROWS_PER_W = _NS_ROWS // _NW
_R = 128
_CHUNKS = _ROWS_PER_W // _R
_NBUF = 6


@functools.partial(
    pl.kernel,
    out_type=jax.ShapeDtypeStruct((_NS_ROWS, _C), jnp.float32),
    mesh=plsc.VectorSubcoreMesh(core_axis_name="c", subcore_axis_name="s"),
    compiler_params=pltpu.CompilerParams(needs_layout_passes=False),
    scratch_types=(
        [pltpu.VMEM((_R, _C), jnp.float32) for _ in range(_NBUF)]
        + [
            pltpu.VMEM((_ROWS_PER_W,), jnp.int32),
            pltpu.VMEM((_ROWS_PER_W,), jnp.int32),
            pltpu.VMEM((_L,), jnp.int32),
        ]
        + [pltpu.SemaphoreType.DMA for _ in range(2 * _NBUF)]
    ),
)
def _sc_dispatch(tensor_2d, op_hbm, pred_hbm, wp_hbm, out_2d,
                 b0, b1, b2, b3, b4, b5, opbuf, prbuf, wpv,
                 is0, is1, is2, is3, is4, is5,
                 os0, os1, os2, os3, os4, os5):
    bufs = [b0, b1, b2, b3, b4, b5]
    isems = [is0, is1, is2, is3, is4, is5]
    osems = [os0, os1, os2, os3, os4, os5]

    wid = lax.axis_index("s") * _NC + lax.axis_index("c")
    base = wid * _ROWS_PER_W

    pltpu.sync_copy(wp_hbm, wpv)
    pltpu.sync_copy(op_hbm.at[pl.ds(base, _ROWS_PER_W)], opbuf)
    pltpu.sync_copy(pred_hbm.at[pl.ds(base, _ROWS_PER_W)], prbuf)
    wp_vec = wpv[...]

    def start_in(g):
        s = g % _NBUF
        return pltpu.async_copy(
            tensor_2d.at[pl.ds(base + g * _R, _R)], bufs[s], isems[s])

    def start_out(g):
        s = g % _NBUF
        return pltpu.async_copy(
            bufs[s], out_2d.at[pl.ds(base + g * _R, _R)], osems[s])

    def compute(g):
        s = g % _NBUF
        buf = bufs[s]
        lbase = g * _R

        def vec_body(j, c2):
            opv = opbuf[pl.ds(lbase + j * _L, _L)]
            prv = prbuf[pl.ds(lbase + j * _L, _L)].astype(jnp.float32)
            pos = lax.gather(
                wp_vec, opv[:, None],
                lax.GatherDimensionNumbers(
                    offset_dims=(), collapsed_slice_dims=(0,),
                    start_index_map=(0,)),
                slice_sizes=(1,),
                mode=lax.GatherScatterMode.PROMISE_IN_BOUNDS)
            rows = lax.iota(jnp.int32, _L) + j * _L
            msk = opv < _NUM_WRITE_OPS
            plsc.store_scatter(buf, [rows, pos], prv, mask=msk)
            return c2

        lax.fori_loop(0, _R // _L, vec_body, 0)

    in_d = {}
    out_d = {}
    for g in range(min(_NBUF, _CHUNKS)):
        in_d[g] = start_in(g)
    for g in range(_CHUNKS):
        in_d[g].wait()
        compute(g)
        out_d[g] = start_out(g)
        if g + _NBUF - 1 < _CHUNKS and g >= 1:
            out_d[g - 1].wait()
            in_d[g + _NBUF - 1] = start_in(g + _NBUF - 1)
    for g in range(max(0, _CHUNKS - _NBUF), _CHUNKS):
        out_d[g].wait()


def kernel(tensor, operation, prediction, write_positions):
    outs = []
    for k in range(_K):
        sl = slice(k * _NS_ROWS, (k + 1) * _NS_ROWS)
        outs.append(_sc_dispatch(tensor[sl], operation[sl],
                                 prediction[sl], write_positions))
    return jnp.concatenate(outs, axis=0)


# speedup vs baseline: 1.3689x; 1.3689x over previous
import functools

import jax
import jax.numpy as jnp
from jax import lax
from jax.experimental import pallas as pl
from jax.experimental.pallas import tpu as pltpu
from jax.experimental.pallas import tpu_sc as plsc

_N = 262144
_C = 64
_NUM_WRITE_OPS = 8

_NC = 2
_NS = 16
_NW = _NC * _NS
_L = 16

_ROWS_PER_W = _N // _NW
_R = 128
_CHUNKS = _ROWS_PER_W // _R
_NBUF = 6


@functools.partial(
    pl.kernel,
    out_type=jax.ShapeDtypeStruct((_N, _C), jnp.float32),
    mesh=plsc.VectorSubcoreMesh(core_axis_name="c", subcore_axis_name="s"),
    compiler_params=pltpu.CompilerParams(needs_layout_passes=False),
    scratch_types=(
        [pltpu.VMEM((_R, _C), jnp.float32) for _ in range(_NBUF)]
        + [
            pltpu.VMEM((_ROWS_PER_W,), jnp.int32),
            pltpu.VMEM((_ROWS_PER_W,), jnp.int32),
            pltpu.VMEM((_L,), jnp.int32),
        ]
        + [pltpu.SemaphoreType.DMA for _ in range(2 * _NBUF)]
    ),
)
def _sc_dispatch(tensor_2d, op_hbm, pred_hbm, wp_hbm, out_2d,
                 b0, b1, b2, b3, b4, b5, opbuf, prbuf, wpv,
                 is0, is1, is2, is3, is4, is5,
                 os0, os1, os2, os3, os4, os5):
    bufs = [b0, b1, b2, b3, b4, b5]
    isems = [is0, is1, is2, is3, is4, is5]
    osems = [os0, os1, os2, os3, os4, os5]

    wid = lax.axis_index("s") * _NC + lax.axis_index("c")
    base = wid * _ROWS_PER_W

    pltpu.sync_copy(wp_hbm, wpv)
    pltpu.sync_copy(op_hbm.at[pl.ds(base, _ROWS_PER_W)], opbuf)
    pltpu.sync_copy(pred_hbm.at[pl.ds(base, _ROWS_PER_W)], prbuf)
    wp_vec = wpv[...]

    def start_in(g):
        s = g % _NBUF
        return pltpu.async_copy(
            tensor_2d.at[pl.ds(base + g * _R, _R)], bufs[s], isems[s])

    def start_out(g):
        s = g % _NBUF
        return pltpu.async_copy(
            bufs[s], out_2d.at[pl.ds(base + g * _R, _R)], osems[s])

    def compute(g):
        s = g % _NBUF
        buf = bufs[s]
        lbase = g * _R

        def vec_body(j, c2):
            opv = opbuf[pl.ds(lbase + j * _L, _L)]
            prv = prbuf[pl.ds(lbase + j * _L, _L)].astype(jnp.float32)
            pos = lax.gather(
                wp_vec, opv[:, None],
                lax.GatherDimensionNumbers(
                    offset_dims=(), collapsed_slice_dims=(0,),
                    start_index_map=(0,)),
                slice_sizes=(1,),
                mode=lax.GatherScatterMode.PROMISE_IN_BOUNDS)
            rows = lax.iota(jnp.int32, _L) + j * _L
            msk = opv < _NUM_WRITE_OPS
            plsc.store_scatter(buf, [rows, pos], prv, mask=msk)
            return c2

        lax.fori_loop(0, _R // _L, vec_body, 0)

    in_d = {}
    out_d = {}
    for g in range(min(_NBUF, _CHUNKS)):
        in_d[g] = start_in(g)
    for g in range(_CHUNKS):
        in_d[g].wait()
        compute(g)
        out_d[g] = start_out(g)
        if g + _NBUF - 1 < _CHUNKS and g >= 1:
            out_d[g - 1].wait()
            in_d[g + _NBUF - 1] = start_in(g + _NBUF - 1)
    for g in range(max(0, _CHUNKS - _NBUF), _CHUNKS):
        out_d[g].wait()


def kernel(tensor, operation, prediction, write_positions):
    return _sc_dispatch(tensor, operation, prediction, write_positions)
